# Initial kernel scaffold; baseline (speedup 1.0000x reference)
import jax
import jax.numpy as jnp
from jax.experimental import pallas as pl


def kernel(inputs, embedding):
    raise NotImplementedError("write your pallas kernel here")


# speedup vs baseline: 4.0926x; 4.0926x over previous
import functools

import jax
import jax.numpy as jnp
from jax import lax
from jax.experimental import pallas as pl
from jax.experimental.pallas import tpu as pltpu
from jax.experimental.pallas import tpu_sc as plsc

D = 64
BATCH = 4096
SEQ = 50
TOKENS = BATCH * SEQ
NC = 2
NS = 16
NW = NC * NS
PER_TILE = TOKENS // NW
CHUNK = 128
CHUNKS = PER_TILE // CHUNK

_mesh = plsc.VectorSubcoreMesh(core_axis_name="c", subcore_axis_name="s")


@functools.partial(
    pl.kernel,
    mesh=_mesh,
    out_type=jax.ShapeDtypeStruct((NW * CHUNKS, CHUNK, D), jnp.float32),
    scratch_types=[
        pltpu.VMEM((CHUNKS, CHUNK), jnp.int32),
        pltpu.VMEM((CHUNK, D), jnp.float32),
        pltpu.SemaphoreType.DMA,
    ],
    compiler_params=pltpu.CompilerParams(use_tc_tiling_on_sc=False),
)
def _gather(idx_hbm, table_hbm, out_hbm, idx_v, rows_v, sem):
    wid = lax.axis_index("s") * NC + lax.axis_index("c")
    pltpu.sync_copy(idx_hbm.at[wid], idx_v)

    def body(j, carry):
        pltpu.async_copy(table_hbm.at[idx_v.at[j]], rows_v, sem).wait()
        pltpu.sync_copy(rows_v, out_hbm.at[wid * CHUNKS + j])
        return carry

    lax.fori_loop(0, CHUNKS, body, 0)


def kernel(inputs, embedding):
    idx = inputs.reshape(NW, CHUNKS, CHUNK)
    out = _gather(idx, embedding)
    return out.reshape(BATCH, SEQ, D)


# speedup vs baseline: 4.6593x; 1.1385x over previous
import functools

import jax
import jax.numpy as jnp
from jax import lax
from jax.experimental import pallas as pl
from jax.experimental.pallas import tpu as pltpu
from jax.experimental.pallas import tpu_sc as plsc

D = 64
BATCH = 4096
SEQ = 50
TOKENS = BATCH * SEQ
NC = 2
NS = 16
NW = NC * NS
PER_TILE = TOKENS // NW
CHUNK = 128
CHUNKS = PER_TILE // CHUNK
NBUF = 10
NGROUPS = CHUNKS // NBUF

_mesh = plsc.VectorSubcoreMesh(core_axis_name="c", subcore_axis_name="s")


@functools.partial(
    pl.kernel,
    mesh=_mesh,
    out_type=jax.ShapeDtypeStruct((NW * CHUNKS, CHUNK, D), jnp.float32),
    scratch_types=[
        pltpu.VMEM((CHUNKS, CHUNK), jnp.int32),
        pltpu.VMEM((NBUF, CHUNK, D), jnp.float32),
        pltpu.SemaphoreType.DMA((NBUF,)),
        pltpu.SemaphoreType.DMA((NBUF,)),
    ],
    compiler_params=pltpu.CompilerParams(use_tc_tiling_on_sc=False),
)
def _gather(idx_hbm, table_hbm, out_hbm, idx_v, rows_v, sem_g, sem_o):
    wid = lax.axis_index("s") * NC + lax.axis_index("c")
    pltpu.sync_copy(idx_hbm.at[wid], idx_v)
    obase = wid * CHUNKS

    def group(g, carry):
        j0 = g * NBUF
        gathers = []
        for b in range(NBUF):
            @pl.when(g > 0)
            def _reclaim(b=b):
                pltpu.make_async_copy(
                    rows_v.at[b], out_hbm.at[obase], sem_o.at[b]
                ).wait()

            gathers.append(
                pltpu.async_copy(
                    table_hbm.at[idx_v.at[j0 + b]], rows_v.at[b], sem_g.at[b]
                )
            )
        for b in range(NBUF):
            gathers[b].wait()
            pltpu.async_copy(rows_v.at[b], out_hbm.at[obase + j0 + b], sem_o.at[b])
        return carry

    lax.fori_loop(0, NGROUPS, group, 0)

    for b in range(NBUF):
        pltpu.make_async_copy(rows_v.at[b], out_hbm.at[obase], sem_o.at[b]).wait()


def kernel(inputs, embedding):
    idx = inputs.reshape(NW, CHUNKS, CHUNK)
    out = _gather(idx, embedding)
    return out.reshape(BATCH, SEQ, D)
